# baseline (speedup 1.0000x reference)
import functools

import jax
import jax.numpy as jnp
from jax import lax
from jax.experimental import pallas as pl
from jax.experimental.pallas import tpu as pltpu
from jax.experimental.pallas import tpu_sc as plsc

_BEAM = 32
_VOCAB = 1_000_000
_ROWS = 8192
_PAD = _ROWS * 128 - _VOCAB
_LANE = 128
_NCH = 256
_CHS = 32
_TOPK = 64
_NLAYER = 12
_KV_ROWS = 512
_D_KV = 128
_BIG = 2**30


_RPB = 2


def _row_topk_kernel(lrow_ref, vals_ref, idx_ref, lse_ref):
    subi = lax.broadcasted_iota(jnp.int32, (_NCH, 1), 0)
    pos2 = (lax.broadcasted_iota(jnp.int32, (_CHS, _LANE), 0) * _LANE
            + lax.broadcasted_iota(jnp.int32, (_CHS, _LANE), 1))
    kio = lax.broadcasted_iota(jnp.int32, (1, _TOPK), 1)

    states = []
    lses = []
    for r in range(_RPB):
        x = lrow_ref[r]
        y = x.reshape(_NCH, _CHS, _LANE)
        colmax = jnp.max(y, axis=1)
        cmax = jnp.max(colmax, axis=1, keepdims=True)
        rowmax = jnp.max(cmax)
        lses.append(rowmax + jnp.log(jnp.sum(jnp.exp(x - rowmax))))
        states.append((cmax, jnp.full((_NCH, 1), -1, jnp.int32),
                       jnp.full((1, _TOPK), -jnp.inf, jnp.float32),
                       jnp.zeros((1, _TOPK), jnp.int32)))

    def body(i, st):
        nxt = []
        for r in range(_RPB):
            cm, lastp, vals, idxs = st[r]
            m = jnp.max(cm, axis=(0, 1), keepdims=True)
            c = jnp.min(jnp.where(cm == m, subi, _BIG))
            lp_c = jnp.sum(jnp.where(subi == c, lastp, 0),
                           axis=(0, 1), keepdims=True)
            chunk = lrow_ref[r, pl.ds(c * _CHS, _CHS), :]
            eq = chunk == m
            valid = eq & (pos2 > lp_c)
            p = jnp.min(jnp.where(valid, pos2, _BIG),
                        axis=(0, 1), keepdims=True)
            cnt = jnp.sum(valid.astype(jnp.int32), axis=(0, 1), keepdims=True)
            mx2 = jnp.max(jnp.where(chunk < m, chunk, -jnp.inf),
                          axis=(0, 1), keepdims=True)
            g = c * (_CHS * _LANE) + p
            vals = jnp.where(kio == i, m, vals)
            idxs = jnp.where(kio == i, g, idxs)
            dup = cnt > 1
            cm = jnp.where(subi == c, jnp.where(dup, m, mx2), cm)
            lastp = jnp.where(subi == c, jnp.where(dup, p, -1), lastp)
            nxt.append((cm, lastp, vals, idxs))
        return tuple(nxt)

    st = lax.fori_loop(0, _TOPK, body, tuple(states))
    for r in range(_RPB):
        _, _, vals, idxs = st[r]
        vals_ref[r] = vals
        idx_ref[r] = idxs
        lse_ref[r] = jnp.full((1, _TOPK), lses[r], jnp.float32)


def _merge_kernel(vals_ref, idx_ref, lse_ref, prev_ref, save_ref,
                  prob_ref, tbi_ref, bidx_ref, nsave_ref):
    cand = vals_ref[...] - lse_ref[...] + prev_ref[...]
    flati = idx_ref[...]
    pos2 = (lax.broadcasted_iota(jnp.int32, (_BEAM, _TOPK), 0) * _TOPK
            + lax.broadcasted_iota(jnp.int32, (_BEAM, _TOPK), 1))
    rio = lax.broadcasted_iota(jnp.int32, (_BEAM, 1), 0)

    def body(i, st):
        cd, probs, tbis, bidxs = st
        m = jnp.max(cd, axis=(0, 1), keepdims=True)
        p = jnp.min(jnp.where(cd == m, pos2, _BIG), axis=(0, 1), keepdims=True)
        tb = jnp.sum(jnp.where(pos2 == p, flati, 0), axis=(0, 1), keepdims=True)
        probs = jnp.where(rio == i, m, probs)
        tbis = jnp.where(rio == i, tb, tbis)
        bidxs = jnp.where(rio == i, p // _TOPK, bidxs)
        cd = jnp.where(pos2 == p, -jnp.inf, cd)
        return cd, probs, tbis, bidxs

    _, probs, tbis, bidxs = lax.fori_loop(
        0, _BEAM, body,
        (cand, jnp.zeros((_BEAM, 1), jnp.float32),
         jnp.zeros((_BEAM, 1), jnp.int32), jnp.zeros((_BEAM, 1), jnp.int32)))
    prob_ref[...] = probs
    tbi_ref[...] = tbis
    bidx_ref[...] = bidxs
    def gbody(j, gs):
        row = save_ref[pl.ds(j, 1), :]
        return jnp.where(bidxs == j, row, gs)

    gs = lax.fori_loop(0, _BEAM, gbody,
                       jnp.zeros((_BEAM, 16), jnp.int32))
    nsave_ref[:, :16] = gs
    nsave_ref[:, 16:] = tbis


def _topk_rows(logits3):
    return pl.pallas_call(
        _row_topk_kernel,
        grid=(_BEAM // _RPB,),
        in_specs=[pl.BlockSpec((_RPB, _ROWS, _LANE), lambda i: (i, 0, 0))],
        out_specs=[pl.BlockSpec((_RPB, 1, _TOPK), lambda i: (i, 0, 0))] * 3,
        out_shape=[
            jax.ShapeDtypeStruct((_BEAM, 1, _TOPK), jnp.float32),
            jax.ShapeDtypeStruct((_BEAM, 1, _TOPK), jnp.int32),
            jax.ShapeDtypeStruct((_BEAM, 1, _TOPK), jnp.float32),
        ],
    )(logits3)


def _merge(vals, idxs, lse, prev, save_id):
    return pl.pallas_call(
        _merge_kernel,
        out_shape=[
            jax.ShapeDtypeStruct((_BEAM, 1), jnp.float32),
            jax.ShapeDtypeStruct((_BEAM, 1), jnp.int32),
            jax.ShapeDtypeStruct((_BEAM, 1), jnp.int32),
            jax.ShapeDtypeStruct((_BEAM, 17), jnp.int32),
        ],
    )(vals, idxs, lse, prev, save_id)


_NC = 2
_NW = 32
_GROWS = _BEAM * 8
_GSUB = _KV_ROWS // 8
_RPW = _GROWS // _NW


def _kv_gather(flat_idx, *kvs3d):
    mesh = plsc.VectorSubcoreMesh(core_axis_name="c", subcore_axis_name="s")

    @functools.partial(
        pl.kernel, mesh=mesh,
        out_type=[jax.ShapeDtypeStruct((_GROWS, _GSUB, _D_KV), jnp.float32)
                  ] * _NLAYER,
        scratch_types=[
            pltpu.VMEM((_RPW,), jnp.int32),
            pltpu.VMEM((_RPW, _GSUB, _D_KV), jnp.float32),
            pltpu.SemaphoreType.DMA,
        ],
    )
    def _gather(idx_hbm, *refs):
        kv_in = refs[:_NLAYER]
        kv_out = refs[_NLAYER:2 * _NLAYER]
        idx_v, buf_v, sem = refs[2 * _NLAYER:]
        wid = lax.axis_index("s") * _NC + lax.axis_index("c")
        base = wid * _RPW
        pltpu.sync_copy(idx_hbm.at[pl.ds(base, _RPW)], idx_v)
        for kv, out in zip(kv_in, kv_out):
            pltpu.async_copy(kv.at[idx_v], buf_v, sem).wait()
            pltpu.sync_copy(buf_v, out.at[pl.ds(base, _RPW)])

    return _gather(flat_idx, *kvs3d)


def kernel(kv_0, kv_1, kv_2, kv_3, kv_4, kv_5, kv_6, kv_7, kv_8, kv_9,
           kv_10, kv_11, logits, save_id, previous_prob, beam_size, top_k):
    kvs = [kv_0, kv_1, kv_2, kv_3, kv_4, kv_5, kv_6, kv_7, kv_8, kv_9,
           kv_10, kv_11]
    lp = jnp.pad(logits, ((0, 0), (0, _PAD)), constant_values=-jnp.inf)
    vals3, idx3, lse3 = _topk_rows(lp.reshape(_BEAM, _ROWS, _LANE))
    probs, tbis, bidxs, nsave = _merge(
        vals3.reshape(_BEAM, _TOPK), idx3.reshape(_BEAM, _TOPK),
        lse3.reshape(_BEAM, _TOPK), previous_prob, save_id)
    flat_idx = (bidxs * 8 + jnp.arange(8, dtype=jnp.int32)[None, :]
                ).reshape(-1)
    outs = _kv_gather(flat_idx,
                      *[kv.reshape(_GROWS, _GSUB, _D_KV) for kv in kvs])
    saved = [o.reshape(_BEAM, _KV_ROWS, _D_KV) for o in outs]
    return (*saved, nsave, probs, tbis, tbis[0:1])


# speedup vs baseline: 1.1387x; 1.1387x over previous
import functools

import jax
import jax.numpy as jnp
from jax import lax
from jax.experimental import pallas as pl
from jax.experimental.pallas import tpu as pltpu
from jax.experimental.pallas import tpu_sc as plsc

_BEAM = 32
_VOCAB = 1_000_000
_ROWS = 8192
_PAD = _ROWS * 128 - _VOCAB
_LANE = 128
_NCH = 64
_CHS = 128
_TOPK = 64
_NLAYER = 12
_KV_ROWS = 512
_D_KV = 128
_BIG = 2**30


_RPB = 4


def _row_topk_kernel(lrow_ref, vals_ref, idx_ref, lse_ref):
    subi = lax.broadcasted_iota(jnp.int32, (_NCH, 1), 0)
    pos2 = (lax.broadcasted_iota(jnp.int32, (_CHS, _LANE), 0) * _LANE
            + lax.broadcasted_iota(jnp.int32, (_CHS, _LANE), 1))
    kio = lax.broadcasted_iota(jnp.int32, (1, _TOPK), 1)

    states = []
    lses = []
    for r in range(_RPB):
        x = lrow_ref[r]
        y = x.reshape(_NCH, _CHS, _LANE)
        colmax = jnp.max(y, axis=1)
        cmax = jnp.max(colmax, axis=1, keepdims=True)
        rowmax = jnp.max(cmax)
        lses.append(rowmax + jnp.log(jnp.sum(jnp.exp(x - rowmax))))
        states.append((cmax, jnp.full((_NCH, 1), -1, jnp.int32),
                       jnp.full((1, _TOPK), -jnp.inf, jnp.float32),
                       jnp.zeros((1, _TOPK), jnp.int32)))

    def body(i, st):
        nxt = []
        for r in range(_RPB):
            cm, lastp, vals, idxs = st[r]
            m = jnp.max(cm, axis=(0, 1), keepdims=True)
            c = jnp.min(jnp.where(cm == m, subi, _BIG))
            lp_c = jnp.sum(jnp.where(subi == c, lastp, 0),
                           axis=(0, 1), keepdims=True)
            chunk = lrow_ref[r, pl.ds(c * _CHS, _CHS), :]
            eq = chunk == m
            valid = eq & (pos2 > lp_c)
            p = jnp.min(jnp.where(valid, pos2, _BIG),
                        axis=(0, 1), keepdims=True)
            cnt = jnp.sum(valid.astype(jnp.int32), axis=(0, 1), keepdims=True)
            mx2 = jnp.max(jnp.where(chunk < m, chunk, -jnp.inf),
                          axis=(0, 1), keepdims=True)
            g = c * (_CHS * _LANE) + p
            vals = jnp.where(kio == i, m, vals)
            idxs = jnp.where(kio == i, g, idxs)
            dup = cnt > 1
            cm = jnp.where(subi == c, jnp.where(dup, m, mx2), cm)
            lastp = jnp.where(subi == c, jnp.where(dup, p, -1), lastp)
            nxt.append((cm, lastp, vals, idxs))
        return tuple(nxt)

    st = lax.fori_loop(0, _TOPK, body, tuple(states))
    for r in range(_RPB):
        _, _, vals, idxs = st[r]
        vals_ref[r] = vals
        idx_ref[r] = idxs
        lse_ref[r] = jnp.full((1, _TOPK), lses[r], jnp.float32)


def _merge_kernel(vals_ref, idx_ref, lse_ref, prev_ref, save_ref,
                  prob_ref, tbi_ref, bidx_ref, nsave_ref):
    cand = vals_ref[...] - lse_ref[...] + prev_ref[...]
    flati = idx_ref[...]
    pos2 = (lax.broadcasted_iota(jnp.int32, (_BEAM, _TOPK), 0) * _TOPK
            + lax.broadcasted_iota(jnp.int32, (_BEAM, _TOPK), 1))
    rio = lax.broadcasted_iota(jnp.int32, (_BEAM, 1), 0)

    def body(i, st):
        cd, probs, tbis, bidxs = st
        m = jnp.max(cd, axis=(0, 1), keepdims=True)
        p = jnp.min(jnp.where(cd == m, pos2, _BIG), axis=(0, 1), keepdims=True)
        tb = jnp.sum(jnp.where(pos2 == p, flati, 0), axis=(0, 1), keepdims=True)
        probs = jnp.where(rio == i, m, probs)
        tbis = jnp.where(rio == i, tb, tbis)
        bidxs = jnp.where(rio == i, p // _TOPK, bidxs)
        cd = jnp.where(pos2 == p, -jnp.inf, cd)
        return cd, probs, tbis, bidxs

    _, probs, tbis, bidxs = lax.fori_loop(
        0, _BEAM, body,
        (cand, jnp.zeros((_BEAM, 1), jnp.float32),
         jnp.zeros((_BEAM, 1), jnp.int32), jnp.zeros((_BEAM, 1), jnp.int32)))
    prob_ref[...] = probs
    tbi_ref[...] = tbis
    bidx_ref[...] = bidxs
    def gbody(j, gs):
        row = save_ref[pl.ds(j, 1), :]
        return jnp.where(bidxs == j, row, gs)

    gs = lax.fori_loop(0, _BEAM, gbody,
                       jnp.zeros((_BEAM, 16), jnp.int32))
    nsave_ref[:, :16] = gs
    nsave_ref[:, 16:] = tbis


def _topk_rows(logits3):
    return pl.pallas_call(
        _row_topk_kernel,
        grid=(_BEAM // _RPB,),
        in_specs=[pl.BlockSpec((_RPB, _ROWS, _LANE), lambda i: (i, 0, 0))],
        out_specs=[pl.BlockSpec((_RPB, 1, _TOPK), lambda i: (i, 0, 0))] * 3,
        out_shape=[
            jax.ShapeDtypeStruct((_BEAM, 1, _TOPK), jnp.float32),
            jax.ShapeDtypeStruct((_BEAM, 1, _TOPK), jnp.int32),
            jax.ShapeDtypeStruct((_BEAM, 1, _TOPK), jnp.float32),
        ],
    )(logits3)


def _merge(vals, idxs, lse, prev, save_id):
    return pl.pallas_call(
        _merge_kernel,
        out_shape=[
            jax.ShapeDtypeStruct((_BEAM, 1), jnp.float32),
            jax.ShapeDtypeStruct((_BEAM, 1), jnp.int32),
            jax.ShapeDtypeStruct((_BEAM, 1), jnp.int32),
            jax.ShapeDtypeStruct((_BEAM, 17), jnp.int32),
        ],
    )(vals, idxs, lse, prev, save_id)


_NC = 2
_NW = 32
_GROWS = _BEAM * 8
_GSUB = _KV_ROWS // 8
_RPW = _GROWS // _NW


def _kv_gather(flat_idx, *kvs3d):
    mesh = plsc.VectorSubcoreMesh(core_axis_name="c", subcore_axis_name="s")

    @functools.partial(
        pl.kernel, mesh=mesh,
        out_type=[jax.ShapeDtypeStruct((_GROWS, _GSUB, _D_KV), jnp.float32)
                  ] * _NLAYER,
        scratch_types=[
            pltpu.VMEM((_RPW,), jnp.int32),
            pltpu.VMEM((_RPW, _GSUB, _D_KV), jnp.float32),
            pltpu.SemaphoreType.DMA,
        ],
    )
    def _gather(idx_hbm, *refs):
        kv_in = refs[:_NLAYER]
        kv_out = refs[_NLAYER:2 * _NLAYER]
        idx_v, buf_v, sem = refs[2 * _NLAYER:]
        wid = lax.axis_index("s") * _NC + lax.axis_index("c")
        base = wid * _RPW
        pltpu.sync_copy(idx_hbm.at[pl.ds(base, _RPW)], idx_v)
        for kv, out in zip(kv_in, kv_out):
            pltpu.async_copy(kv.at[idx_v], buf_v, sem).wait()
            pltpu.sync_copy(buf_v, out.at[pl.ds(base, _RPW)])

    return _gather(flat_idx, *kvs3d)


def kernel(kv_0, kv_1, kv_2, kv_3, kv_4, kv_5, kv_6, kv_7, kv_8, kv_9,
           kv_10, kv_11, logits, save_id, previous_prob, beam_size, top_k):
    kvs = [kv_0, kv_1, kv_2, kv_3, kv_4, kv_5, kv_6, kv_7, kv_8, kv_9,
           kv_10, kv_11]
    lp = jnp.pad(logits, ((0, 0), (0, _PAD)), constant_values=-jnp.inf)
    vals3, idx3, lse3 = _topk_rows(lp.reshape(_BEAM, _ROWS, _LANE))
    probs, tbis, bidxs, nsave = _merge(
        vals3.reshape(_BEAM, _TOPK), idx3.reshape(_BEAM, _TOPK),
        lse3.reshape(_BEAM, _TOPK), previous_prob, save_id)
    flat_idx = (bidxs * 8 + jnp.arange(8, dtype=jnp.int32)[None, :]
                ).reshape(-1)
    outs = _kv_gather(flat_idx,
                      *[kv.reshape(_GROWS, _GSUB, _D_KV) for kv in kvs])
    saved = [o.reshape(_BEAM, _KV_ROWS, _D_KV) for o in outs]
    return (*saved, nsave, probs, tbis, tbis[0:1])
